# baseline (speedup 1.0000x reference)
import functools

import jax
import jax.numpy as jnp
from jax import lax
from jax.experimental import pallas as pl
from jax.experimental.pallas import tpu as pltpu
from jax.experimental.pallas import tpu_sc as plsc

A, B, C0, D = 16, 16, 16, 128
N = A * B * C0 * 8 * D
NW = 32
CELLS_PER_W = N // NW
CHUNK = 4096
ITERS = CELLS_PER_W // CHUNK
GROUPS = CHUNK // 16


def _body(input_hbm, mean_hbm, sig_hbm, eps_hbm, out_hbm,
          input_v, mean_v, sig_v, eps_v, outp_v, outn_v):
    wid = lax.axis_index("s") * 2 + lax.axis_index("c")
    iota = lax.iota(jnp.int32, 16)
    iota4 = iota * 4

    def chunk_body(it, _):
        base_cell = pl.multiple_of(wid * CELLS_PER_W + it * CHUNK, CHUNK)
        pltpu.sync_copy(
            input_hbm.at[pl.ds(pl.multiple_of(base_cell // 8, CHUNK // 8),
                               CHUNK // 8)],
            input_v)
        pltpu.sync_copy(
            mean_hbm.at[pl.ds(pl.multiple_of(base_cell * 4, CHUNK * 4),
                              CHUNK * 4)],
            mean_v)
        pltpu.sync_copy(
            sig_hbm.at[pl.ds(pl.multiple_of(base_cell * 4, CHUNK * 4),
                             CHUNK * 4)],
            sig_v)
        pltpu.sync_copy(eps_hbm.at[pl.ds(base_cell, CHUNK)], eps_v)

        def group_body(g, _):
            cell_local = g * 16
            x_off = (g >> 6) * 128 + (g & 7) * 16
            shift = 14 - 2 * ((g >> 3) & 7)

            x = input_v[pl.ds(x_off, 16)]
            xp = jnp.maximum(x, 0.0).astype(jnp.int32)
            xn = jnp.maximum(-x, 0.0).astype(jnp.int32)
            cp = (xp >> shift) & 3
            cn = (xn >> shift) & 3
            base4 = cell_local * 4 + iota4
            mp = plsc.load_gather(mean_v, [base4 + cp])
            sp = plsc.load_gather(sig_v, [base4 + cp])
            mn = plsc.load_gather(mean_v, [base4 + cn])
            sn = plsc.load_gather(sig_v, [base4 + cn])
            e = eps_v[pl.ds(cell_local, 16)]
            outp_v[pl.ds(cell_local, 16)] = mp + e * sp
            outn_v[pl.ds(cell_local, 16)] = mn + e * sn
            return 0

        lax.fori_loop(0, GROUPS, group_body, 0)

        pltpu.sync_copy(outp_v, out_hbm.at[pl.ds(base_cell, CHUNK)])
        pltpu.sync_copy(outn_v, out_hbm.at[pl.ds(N + base_cell, CHUNK)])
        return 0

    lax.fori_loop(0, ITERS, chunk_body, 0)


@jax.jit
def kernel(input, mean_G, sig_G, eps):
    mesh = plsc.VectorSubcoreMesh(core_axis_name="c", subcore_axis_name="s",
                                  num_cores=2, num_subcores=16)
    run = functools.partial(
        pl.kernel,
        out_type=jax.ShapeDtypeStruct((2 * N,), jnp.float32),
        mesh=mesh,
        compiler_params=pltpu.CompilerParams(needs_layout_passes=False),
        scratch_types=[
            pltpu.VMEM((CHUNK // 8,), jnp.float32),
            pltpu.VMEM((CHUNK * 4,), jnp.float32),
            pltpu.VMEM((CHUNK * 4,), jnp.float32),
            pltpu.VMEM((CHUNK,), jnp.float32),
            pltpu.VMEM((CHUNK,), jnp.float32),
            pltpu.VMEM((CHUNK,), jnp.float32),
        ],
    )(_body)
    out = run(input.reshape(-1), mean_G.reshape(-1), sig_G.reshape(-1),
              eps.reshape(-1))
    g_inactive = out.reshape(2, A, B, C0 * 8, D)
    g_active = jnp.zeros_like(g_inactive)
    return (g_active, g_inactive)


# speedup vs baseline: 1.0186x; 1.0186x over previous
import functools

import jax
import jax.numpy as jnp
from jax import lax
from jax.experimental import pallas as pl
from jax.experimental.pallas import tpu as pltpu
from jax.experimental.pallas import tpu_sc as plsc

A, B, C0, D = 16, 16, 16, 128
N = A * B * C0 * 8 * D
NW = 32
CELLS_PER_W = N // NW
CHUNK = 4096
ITERS = CELLS_PER_W // CHUNK
GROUPS = CHUNK // 16


def _body(input_hbm, mean_hbm, sig_hbm, eps_hbm, out_hbm,
          input_v0, mean_v0, sig_v0, eps_v0, outp_v0, outn_v0,
          input_v1, mean_v1, sig_v1, eps_v1, outp_v1, outn_v1,
          sem_in, sem_out):
    wid = lax.axis_index("s") * 2 + lax.axis_index("c")
    bufs = ((input_v0, mean_v0, sig_v0, eps_v0, outp_v0, outn_v0),
            (input_v1, mean_v1, sig_v1, eps_v1, outp_v1, outn_v1))
    iota = lax.iota(jnp.int32, 16)
    iota4 = iota * 4

    def base_of(it):
        return pl.multiple_of(wid * CELLS_PER_W + it * CHUNK, CHUNK)

    def in_copies(it, b):
        base = base_of(it)
        iv, mv, sv, ev, _, _ = bufs[b]
        return [
            pltpu.make_async_copy(
                input_hbm.at[pl.ds(pl.multiple_of(base // 8, CHUNK // 8),
                                   CHUNK // 8)],
                iv, sem_in.at[b]),
            pltpu.make_async_copy(
                mean_hbm.at[pl.ds(pl.multiple_of(base * 4, CHUNK * 4),
                                  CHUNK * 4)],
                mv, sem_in.at[b]),
            pltpu.make_async_copy(
                sig_hbm.at[pl.ds(pl.multiple_of(base * 4, CHUNK * 4),
                                 CHUNK * 4)],
                sv, sem_in.at[b]),
            pltpu.make_async_copy(
                eps_hbm.at[pl.ds(base, CHUNK)], ev, sem_in.at[b]),
        ]

    def out_copies(it, b):
        base = base_of(it)
        ov_p, ov_n = bufs[b][4], bufs[b][5]
        return [
            pltpu.make_async_copy(
                ov_p, out_hbm.at[pl.ds(base, CHUNK)], sem_out.at[b]),
            pltpu.make_async_copy(
                ov_n, out_hbm.at[pl.ds(N + base, CHUNK)],
                sem_out.at[b]),
        ]

    def compute(b):
        iv, mv, sv, ev, ov_p, ov_n = bufs[b]

        def _t_body(t, _):
            row = t >> 3
            d0 = (t & 7) * 16
            x = iv[pl.ds(row * 128 + d0, 16)]
            xp = jnp.maximum(x, 0.0).astype(jnp.int32)
            xn = jnp.maximum(-x, 0.0).astype(jnp.int32)
            cell0 = row * 1024 + d0
            base4 = cell0 * 4 + iota4
            for k in range(8):
                shift = 14 - 2 * k
                cp = (xp >> shift) & 3
                cn = (xn >> shift) & 3
                bk = base4 + k * 512
                idxp = bk + cp
                idxn = bk + cn
                mp = plsc.load_gather(mv, [idxp])
                sp = plsc.load_gather(sv, [idxp])
                mn = plsc.load_gather(mv, [idxn])
                sn = plsc.load_gather(sv, [idxn])
                off = cell0 + k * 128
                e = ev[pl.ds(off, 16)]
                ov_p[pl.ds(off, 16)] = mp + e * sp
                ov_n[pl.ds(off, 16)] = mn + e * sn
            return 0

        lax.fori_loop(0, (CHUNK // 1024) * 8, _t_body, 0)

    for b in range(2):
        for c in in_copies(b, b):
            c.start()

    def loop2(i2, _):
        for b in range(2):
            it = i2 * 2 + b
            for c in in_copies(it, b):
                c.wait()

            @pl.when(it >= 2)
            def _():
                for c in out_copies(it - 2, b):
                    c.wait()

            compute(b)
            for c in out_copies(it, b):
                c.start()

            @pl.when(it + 2 < ITERS)
            def _():
                for c in in_copies(it + 2, b):
                    c.start()
        return 0

    lax.fori_loop(0, ITERS // 2, loop2, 0)

    for b in range(2):
        for c in out_copies(ITERS - 2 + b, b):
            c.wait()


@jax.jit
def kernel(input, mean_G, sig_G, eps):
    mesh = plsc.VectorSubcoreMesh(core_axis_name="c", subcore_axis_name="s",
                                  num_cores=2, num_subcores=16)
    run = functools.partial(
        pl.kernel,
        out_type=jax.ShapeDtypeStruct((2 * N,), jnp.float32),
        mesh=mesh,
        compiler_params=pltpu.CompilerParams(needs_layout_passes=False),
        scratch_types=(
            [pltpu.VMEM((CHUNK // 8,), jnp.float32),
             pltpu.VMEM((CHUNK * 4,), jnp.float32),
             pltpu.VMEM((CHUNK * 4,), jnp.float32),
             pltpu.VMEM((CHUNK,), jnp.float32),
             pltpu.VMEM((CHUNK,), jnp.float32),
             pltpu.VMEM((CHUNK,), jnp.float32)] * 2
            + [pltpu.SemaphoreType.DMA((2,)),
               pltpu.SemaphoreType.DMA((2,))]),
    )(_body)
    out = run(input.reshape(-1), mean_G.reshape(-1), sig_G.reshape(-1),
              eps.reshape(-1))
    g_inactive = out.reshape(2, A, B, C0 * 8, D)
    g_active = jnp.zeros_like(g_inactive)
    return (g_active, g_inactive)


# speedup vs baseline: 93.0657x; 91.3644x over previous
import functools

import jax
import jax.numpy as jnp
from jax import lax
from jax.experimental import pallas as pl
from jax.experimental.pallas import tpu as pltpu
from jax.experimental.pallas import tpu_sc as plsc

A, B, C0, D = 16, 16, 16, 128
N = A * B * C0 * 8 * D
NW = 32
CELLS_PER_W = N // NW
CHUNK = 4096
ITERS = CELLS_PER_W // CHUNK
GROUPS = CHUNK // 16


def _body(input_hbm, mean_hbm, sig_hbm, eps_hbm, out_hbm,
          input_v0, mean_v0, sig_v0, eps_v0, outp_v0, outn_v0,
          input_v1, mean_v1, sig_v1, eps_v1, outp_v1, outn_v1,
          sem_in, sem_out):
    wid = lax.axis_index("s") * 2 + lax.axis_index("c")
    bufs = ((input_v0, mean_v0, sig_v0, eps_v0, outp_v0, outn_v0),
            (input_v1, mean_v1, sig_v1, eps_v1, outp_v1, outn_v1))
    iota = lax.iota(jnp.int32, 16)

    def base_of(it):
        return pl.multiple_of(wid * CELLS_PER_W + it * CHUNK, CHUNK)

    def in_copies(it, b):
        base = base_of(it)
        iv, mv, sv, ev, _, _ = bufs[b]
        return [
            pltpu.make_async_copy(
                input_hbm.at[pl.ds(pl.multiple_of(base // 8, CHUNK // 8),
                                   CHUNK // 8)],
                iv, sem_in.at[b]),
            pltpu.make_async_copy(
                mean_hbm.at[pl.ds(pl.multiple_of(base * 4, CHUNK * 4),
                                  CHUNK * 4)],
                mv, sem_in.at[b]),
            pltpu.make_async_copy(
                sig_hbm.at[pl.ds(pl.multiple_of(base * 4, CHUNK * 4),
                                 CHUNK * 4)],
                sv, sem_in.at[b]),
            pltpu.make_async_copy(
                eps_hbm.at[pl.ds(base, CHUNK)], ev, sem_in.at[b]),
        ]

    def out_copies(it, b):
        base = base_of(it)
        ov_p, ov_n = bufs[b][4], bufs[b][5]
        return [
            pltpu.make_async_copy(
                ov_p, out_hbm.at[pl.ds(base, CHUNK)], sem_out.at[b]),
            pltpu.make_async_copy(
                ov_n, out_hbm.at[pl.ds(N + base, CHUNK)],
                sem_out.at[b]),
        ]

    def compute(b):
        iv, mv, sv, ev, ov_p, ov_n = bufs[b]

        def _t_body(t, _):
            row = t >> 3
            d0 = (t & 7) * 16
            x = iv[pl.ds(row * 128 + d0, 16)]
            xp = jnp.maximum(x, 0.0).astype(jnp.int32)
            xn = jnp.maximum(-x, 0.0).astype(jnp.int32)
            cell0 = row * 1024 + d0
            vec = (row * 4096 + d0) + iota
            for k in range(8):
                shift = 14 - 2 * k
                cp = (xp >> shift) & 3
                cn = (xn >> shift) & 3
                veck = vec + k * 512
                idxp = veck + (cp << 7)
                idxn = veck + (cn << 7)
                mp = plsc.load_gather(mv, [idxp])
                sp = plsc.load_gather(sv, [idxp])
                mn = plsc.load_gather(mv, [idxn])
                sn = plsc.load_gather(sv, [idxn])
                off = cell0 + k * 128
                e = ev[pl.ds(off, 16)]
                ov_p[pl.ds(off, 16)] = mp + e * sp
                ov_n[pl.ds(off, 16)] = mn + e * sn
            return 0

        lax.fori_loop(0, (CHUNK // 1024) * 8, _t_body, 0)

    for b in range(2):
        for c in in_copies(b, b):
            c.start()

    def loop2(i2, _):
        for b in range(2):
            it = i2 * 2 + b
            for c in in_copies(it, b):
                c.wait()

            @pl.when(it >= 2)
            def _():
                for c in out_copies(it - 2, b):
                    c.wait()

            compute(b)
            for c in out_copies(it, b):
                c.start()

            @pl.when(it + 2 < ITERS)
            def _():
                for c in in_copies(it + 2, b):
                    c.start()
        return 0

    lax.fori_loop(0, ITERS // 2, loop2, 0)

    for b in range(2):
        for c in out_copies(ITERS - 2 + b, b):
            c.wait()


@jax.jit
def kernel(input, mean_G, sig_G, eps):
    mesh = plsc.VectorSubcoreMesh(core_axis_name="c", subcore_axis_name="s",
                                  num_cores=2, num_subcores=16)
    run = functools.partial(
        pl.kernel,
        out_type=jax.ShapeDtypeStruct((2 * N,), jnp.float32),
        mesh=mesh,
        compiler_params=pltpu.CompilerParams(needs_layout_passes=False),
        scratch_types=(
            [pltpu.VMEM((CHUNK // 8,), jnp.float32),
             pltpu.VMEM((CHUNK * 4,), jnp.float32),
             pltpu.VMEM((CHUNK * 4,), jnp.float32),
             pltpu.VMEM((CHUNK,), jnp.float32),
             pltpu.VMEM((CHUNK,), jnp.float32),
             pltpu.VMEM((CHUNK,), jnp.float32)] * 2
            + [pltpu.SemaphoreType.DMA((2,)),
               pltpu.SemaphoreType.DMA((2,))]),
    )(_body)
    mean_lin = jnp.transpose(mean_G, (0, 1, 2, 4, 3)).reshape(-1)
    sig_lin = jnp.transpose(sig_G, (0, 1, 2, 4, 3)).reshape(-1)
    out = run(input.reshape(-1), mean_lin, sig_lin, eps.reshape(-1))
    g_inactive = out.reshape(2, A, B, C0 * 8, D)
    g_active = jnp.zeros_like(g_inactive)
    return (g_active, g_inactive)


# speedup vs baseline: 98.3731x; 1.0570x over previous
import functools

import jax
import jax.numpy as jnp
from jax import lax
from jax.experimental import pallas as pl
from jax.experimental.pallas import tpu as pltpu
from jax.experimental.pallas import tpu_sc as plsc

A, B, C0, D = 16, 16, 16, 128
N = A * B * C0 * 8 * D
NW = 32
CELLS_PER_W = N // NW
CHUNK = 4096
ITERS = CELLS_PER_W // CHUNK
GROUPS = CHUNK // 16


def _body(input_hbm, mean_hbm, sig_hbm, eps_hbm, out_hbm, outz_hbm,
          input_v0, mean_v0, sig_v0, eps_v0, outp_v0, outn_v0,
          input_v1, mean_v1, sig_v1, eps_v1, outp_v1, outn_v1,
          zero_v, sem_in, sem_out):
    wid = lax.axis_index("s") * 2 + lax.axis_index("c")
    bufs = ((input_v0, mean_v0, sig_v0, eps_v0, outp_v0, outn_v0),
            (input_v1, mean_v1, sig_v1, eps_v1, outp_v1, outn_v1))
    iota = lax.iota(jnp.int32, 16)

    def base_of(it):
        return pl.multiple_of(wid * CELLS_PER_W + it * CHUNK, CHUNK)

    def in_copies(it, b):
        base = base_of(it)
        iv, mv, sv, ev, _, _ = bufs[b]
        return [
            pltpu.make_async_copy(
                input_hbm.at[pl.ds(pl.multiple_of(base // 8, CHUNK // 8),
                                   CHUNK // 8)],
                iv, sem_in.at[b]),
            pltpu.make_async_copy(
                mean_hbm.at[pl.ds(pl.multiple_of(base * 4, CHUNK * 4),
                                  CHUNK * 4)],
                mv, sem_in.at[b]),
            pltpu.make_async_copy(
                sig_hbm.at[pl.ds(pl.multiple_of(base * 4, CHUNK * 4),
                                 CHUNK * 4)],
                sv, sem_in.at[b]),
            pltpu.make_async_copy(
                eps_hbm.at[pl.ds(base, CHUNK)], ev, sem_in.at[b]),
        ]

    def out_copies(it, b):
        base = base_of(it)
        ov_p, ov_n = bufs[b][4], bufs[b][5]
        return [
            pltpu.make_async_copy(
                ov_p, out_hbm.at[pl.ds(base, CHUNK)], sem_out.at[b]),
            pltpu.make_async_copy(
                ov_n, out_hbm.at[pl.ds(N + base, CHUNK)],
                sem_out.at[b]),
            pltpu.make_async_copy(
                zero_v, outz_hbm.at[pl.ds(base, CHUNK)], sem_out.at[b]),
            pltpu.make_async_copy(
                zero_v, outz_hbm.at[pl.ds(N + base, CHUNK)],
                sem_out.at[b]),
        ]

    def compute(b):
        iv, mv, sv, ev, ov_p, ov_n = bufs[b]

        def _t_body(t, _):
            row = t >> 3
            d0 = (t & 7) * 16
            x = iv[pl.ds(row * 128 + d0, 16)]
            xp = jnp.maximum(x, 0.0).astype(jnp.int32)
            xn = jnp.maximum(-x, 0.0).astype(jnp.int32)
            cell0 = row * 1024 + d0
            vec = (row * 4096 + d0) + iota
            for k in range(8):
                shift = 14 - 2 * k
                cp = (xp >> shift) & 3
                cn = (xn >> shift) & 3
                veck = vec + k * 512
                idxp = veck + (cp << 7)
                idxn = veck + (cn << 7)
                mp = plsc.load_gather(mv, [idxp])
                sp = plsc.load_gather(sv, [idxp])
                mn = plsc.load_gather(mv, [idxn])
                sn = plsc.load_gather(sv, [idxn])
                off = cell0 + k * 128
                e = ev[pl.ds(off, 16)]
                ov_p[pl.ds(off, 16)] = mp + e * sp
                ov_n[pl.ds(off, 16)] = mn + e * sn
            return 0

        lax.fori_loop(0, (CHUNK // 1024) * 8, _t_body, 0)

    zvec = jnp.zeros((16,), jnp.float32)

    def zinit(g, _):
        zero_v[pl.ds(g * 16, 16)] = zvec
        return 0

    lax.fori_loop(0, CHUNK // 16, zinit, 0)

    for b in range(2):
        for c in in_copies(b, b):
            c.start()

    def loop2(i2, _):
        for b in range(2):
            it = i2 * 2 + b
            for c in in_copies(it, b):
                c.wait()

            @pl.when(it >= 2)
            def _():
                for c in out_copies(it - 2, b):
                    c.wait()

            compute(b)
            for c in out_copies(it, b):
                c.start()

            @pl.when(it + 2 < ITERS)
            def _():
                for c in in_copies(it + 2, b):
                    c.start()
        return 0

    lax.fori_loop(0, ITERS // 2, loop2, 0)

    for b in range(2):
        for c in out_copies(ITERS - 2 + b, b):
            c.wait()


@jax.jit
def kernel(input, mean_G, sig_G, eps):
    mesh = plsc.VectorSubcoreMesh(core_axis_name="c", subcore_axis_name="s",
                                  num_cores=2, num_subcores=16)
    run = functools.partial(
        pl.kernel,
        out_type=(jax.ShapeDtypeStruct((2 * N,), jnp.float32),
                  jax.ShapeDtypeStruct((2 * N,), jnp.float32)),
        mesh=mesh,
        compiler_params=pltpu.CompilerParams(needs_layout_passes=False),
        scratch_types=(
            [pltpu.VMEM((CHUNK // 8,), jnp.float32),
             pltpu.VMEM((CHUNK * 4,), jnp.float32),
             pltpu.VMEM((CHUNK * 4,), jnp.float32),
             pltpu.VMEM((CHUNK,), jnp.float32),
             pltpu.VMEM((CHUNK,), jnp.float32),
             pltpu.VMEM((CHUNK,), jnp.float32)] * 2
            + [pltpu.VMEM((CHUNK,), jnp.float32),
               pltpu.SemaphoreType.DMA((2,)),
               pltpu.SemaphoreType.DMA((2,))]),
    )(_body)
    mean_lin = jnp.transpose(mean_G, (0, 1, 2, 4, 3)).reshape(-1)
    sig_lin = jnp.transpose(sig_G, (0, 1, 2, 4, 3)).reshape(-1)
    out, outz = run(input.reshape(-1), mean_lin, sig_lin, eps.reshape(-1))
    g_inactive = out.reshape(2, A, B, C0 * 8, D)
    g_active = outz.reshape(2, A, B, C0 * 8, D)
    return (g_active, g_inactive)


# speedup vs baseline: 106.0142x; 1.0777x over previous
import functools

import jax
import jax.numpy as jnp
from jax import lax
from jax.experimental import pallas as pl
from jax.experimental.pallas import tpu as pltpu
from jax.experimental.pallas import tpu_sc as plsc

A, B, C0, D = 16, 16, 16, 128
N = A * B * C0 * 8 * D
NW = 32
CELLS_PER_W = N // NW
CHUNK = 2048
NBUF = 4
ITERS = CELLS_PER_W // CHUNK
T_PER_CHUNK = (CHUNK // 1024) * 8


def _body(input_hbm, mean_hbm, sig_hbm, eps_hbm, out_hbm, outz_hbm, *rest):
    bufs = tuple(tuple(rest[b * 6:(b + 1) * 6]) for b in range(NBUF))
    zero_v, sem_in, sem_out = rest[NBUF * 6:]
    wid = lax.axis_index("s") * 2 + lax.axis_index("c")
    iota = lax.iota(jnp.int32, 16)

    def base_of(it):
        return pl.multiple_of(wid * CELLS_PER_W + it * CHUNK, CHUNK)

    def in_copies(it, b):
        base = base_of(it)
        iv, mv, sv, ev, _, _ = bufs[b]
        return [
            pltpu.make_async_copy(
                input_hbm.at[pl.ds(pl.multiple_of(base // 8, CHUNK // 8),
                                   CHUNK // 8)],
                iv, sem_in.at[b]),
            pltpu.make_async_copy(
                mean_hbm.at[pl.ds(pl.multiple_of(base * 4, CHUNK * 4),
                                  CHUNK * 4)],
                mv, sem_in.at[b]),
            pltpu.make_async_copy(
                sig_hbm.at[pl.ds(pl.multiple_of(base * 4, CHUNK * 4),
                                 CHUNK * 4)],
                sv, sem_in.at[b]),
            pltpu.make_async_copy(
                eps_hbm.at[pl.ds(base, CHUNK)], ev, sem_in.at[b]),
        ]

    def out_copies(it, b):
        base = base_of(it)
        ov_p, ov_n = bufs[b][4], bufs[b][5]
        return [
            pltpu.make_async_copy(
                ov_p, out_hbm.at[pl.ds(base, CHUNK)], sem_out.at[b]),
            pltpu.make_async_copy(
                ov_n, out_hbm.at[pl.ds(N + base, CHUNK)], sem_out.at[b]),
            pltpu.make_async_copy(
                zero_v, outz_hbm.at[pl.ds(base, CHUNK)], sem_out.at[b]),
            pltpu.make_async_copy(
                zero_v, outz_hbm.at[pl.ds(N + base, CHUNK)], sem_out.at[b]),
        ]

    def compute(b):
        iv, mv, sv, ev, ov_p, ov_n = bufs[b]

        def _t_body(t, _):
            row = t >> 3
            d0 = (t & 7) * 16
            x = iv[pl.ds(row * 128 + d0, 16)]
            xp = jnp.maximum(x, 0.0).astype(jnp.int32)
            xn = jnp.maximum(-x, 0.0).astype(jnp.int32)
            cell0 = row * 1024 + d0
            vec = (row * 4096 + d0) + iota
            for k in range(8):
                shift = 14 - 2 * k
                cp = (xp >> shift) & 3
                cn = (xn >> shift) & 3
                veck = vec + k * 512
                idxp = veck + (cp << 7)
                idxn = veck + (cn << 7)
                mp = plsc.load_gather(mv, [idxp])
                sp = plsc.load_gather(sv, [idxp])
                mn = plsc.load_gather(mv, [idxn])
                sn = plsc.load_gather(sv, [idxn])
                off = cell0 + k * 128
                e = ev[pl.ds(off, 16)]
                ov_p[pl.ds(off, 16)] = mp + e * sp
                ov_n[pl.ds(off, 16)] = mn + e * sn
            return 0

        lax.fori_loop(0, T_PER_CHUNK, _t_body, 0)

    zvec = jnp.zeros((16,), jnp.float32)

    def zinit(g, _):
        zero_v[pl.ds(g * 16, 16)] = zvec
        return 0

    lax.fori_loop(0, CHUNK // 16, zinit, 0)

    for b in range(NBUF):
        for c in in_copies(b, b):
            c.start()

    def loop_ring(i, _):
        for b in range(NBUF):
            it = i * NBUF + b
            for c in in_copies(it, b):
                c.wait()

            @pl.when(it >= NBUF)
            def _():
                for c in out_copies(it - NBUF, b):
                    c.wait()

            compute(b)
            for c in out_copies(it, b):
                c.start()

            @pl.when(it + NBUF < ITERS)
            def _():
                for c in in_copies(it + NBUF, b):
                    c.start()
        return 0

    lax.fori_loop(0, ITERS // NBUF, loop_ring, 0)

    for b in range(NBUF):
        for c in out_copies(ITERS - NBUF + b, b):
            c.wait()


@jax.jit
def kernel(input, mean_G, sig_G, eps):
    mesh = plsc.VectorSubcoreMesh(core_axis_name="c", subcore_axis_name="s",
                                  num_cores=2, num_subcores=16)
    run = functools.partial(
        pl.kernel,
        out_type=(jax.ShapeDtypeStruct((2 * N,), jnp.float32),
                  jax.ShapeDtypeStruct((2 * N,), jnp.float32)),
        mesh=mesh,
        compiler_params=pltpu.CompilerParams(needs_layout_passes=False),
        scratch_types=(
            [pltpu.VMEM((CHUNK // 8,), jnp.float32),
             pltpu.VMEM((CHUNK * 4,), jnp.float32),
             pltpu.VMEM((CHUNK * 4,), jnp.float32),
             pltpu.VMEM((CHUNK,), jnp.float32),
             pltpu.VMEM((CHUNK,), jnp.float32),
             pltpu.VMEM((CHUNK,), jnp.float32)] * NBUF
            + [pltpu.VMEM((CHUNK,), jnp.float32),
               pltpu.SemaphoreType.DMA((NBUF,)),
               pltpu.SemaphoreType.DMA((NBUF,))]),
    )(_body)
    mean_lin = jnp.transpose(mean_G, (0, 1, 2, 4, 3)).reshape(-1)
    sig_lin = jnp.transpose(sig_G, (0, 1, 2, 4, 3)).reshape(-1)
    out, outz = run(input.reshape(-1), mean_lin, sig_lin, eps.reshape(-1))
    g_inactive = out.reshape(2, A, B, C0 * 8, D)
    g_active = outz.reshape(2, A, B, C0 * 8, D)
    return (g_active, g_inactive)
